# baseline (speedup 1.0000x reference)
import jax
import jax.numpy as jnp
from jax import lax
from jax.experimental import pallas as pl
from jax.experimental.pallas import tpu as pltpu

NUM_EMB = 1024
DIM = 64
COMMIT = 0.25
TILE_N = 2304


def _vq_body(x_ref, x2_ref, cb_ref, q_ref, idx_ref, loss_ref, perp_ref,
             cb2h_ref, fiota_ref, counts_ref, smaxacc_ref, xsqacc_ref):
    step = pl.program_id(0)
    nsteps = pl.num_programs(0)
    x = x_ref[...]
    x2 = x2_ref[...]
    cb = cb_ref[...]

    @pl.when(step == 0)
    def _():
        cb2h_ref[...] = 0.5 * jnp.sum(cb * cb, axis=1, keepdims=True)
        fiota_ref[...] = lax.broadcasted_iota(
            jnp.int32, (NUM_EMB, 1), 0).astype(jnp.float32)

    xcT = lax.dot_general(cb, x, (((1,), (1,)), ((), ())),
                          preferred_element_type=jnp.float32)
    sT = xcT - cb2h_ref[...]
    smax = jnp.max(sT, axis=0, keepdims=True)
    fiota = fiota_ref[...]
    idx_row = jnp.min(jnp.where(sT == smax, fiota, 131072.0),
                      axis=0, keepdims=True)
    idx_ref[0, :, :] = idx_row.astype(jnp.int32)
    onehotT = (fiota == idx_row).astype(jnp.float32)
    q_ref[...] = lax.dot_general(onehotT, cb, (((0,), (0,)), ((), ())),
                                 preferred_element_type=jnp.float32)
    part_xsq = jnp.sum(x2 * x2, axis=0, keepdims=True)
    ones_col = jnp.ones((TILE_N, 1), jnp.float32)
    part_counts = lax.dot_general(onehotT, ones_col, (((1,), (0,)), ((), ())),
                                  preferred_element_type=jnp.float32)

    @pl.when(step == 0)
    def _():
        counts_ref[...] = part_counts
        smaxacc_ref[...] = smax
        xsqacc_ref[...] = part_xsq

    @pl.when(step != 0)
    def _():
        counts_ref[...] += part_counts
        smaxacc_ref[...] += smax
        xsqacc_ref[...] += part_xsq

    @pl.when(step == nsteps - 1)
    def _():
        n_total = nsteps * TILE_N
        p = counts_ref[...] * (1.0 / n_total)
        perp_ref[0, 0] = jnp.exp(-jnp.sum(p * jnp.log(p + 1e-10)))
        lsum = jnp.sum(xsqacc_ref[...]) - 2.0 * jnp.sum(smaxacc_ref[...])
        loss_ref[0, 0] = (1.0 + COMMIT) * lsum / (n_total * DIM)


def kernel(inputs, codebook):
    flat = inputs.reshape(-1, DIM)
    n = flat.shape[0]
    grid = (n // TILE_N,)
    q, idx3, loss, perp = pl.pallas_call(
        _vq_body,
        grid=grid,
        in_specs=[
            pl.BlockSpec((TILE_N, DIM), lambda i: (i, 0)),
            pl.BlockSpec((TILE_N // 2, 2 * DIM), lambda i: (i, 0)),
            pl.BlockSpec((NUM_EMB, DIM), lambda i: (0, 0)),
        ],
        out_specs=[
            pl.BlockSpec((TILE_N, DIM), lambda i: (i, 0)),
            pl.BlockSpec((1, 1, TILE_N), lambda i: (i, 0, 0)),
            pl.BlockSpec(memory_space=pltpu.SMEM),
            pl.BlockSpec(memory_space=pltpu.SMEM),
        ],
        out_shape=[
            jax.ShapeDtypeStruct((n, DIM), jnp.float32),
            jax.ShapeDtypeStruct((n // TILE_N, 1, TILE_N), jnp.int32),
            jax.ShapeDtypeStruct((1, 1), jnp.float32),
            jax.ShapeDtypeStruct((1, 1), jnp.float32),
        ],
        scratch_shapes=[
            pltpu.VMEM((NUM_EMB, 1), jnp.float32),
            pltpu.VMEM((NUM_EMB, 1), jnp.float32),
            pltpu.VMEM((NUM_EMB, 1), jnp.float32),
            pltpu.VMEM((1, TILE_N), jnp.float32),
            pltpu.VMEM((1, 2 * DIM), jnp.float32),
        ],
        compiler_params=pltpu.CompilerParams(
            dimension_semantics=("arbitrary",)),
    )(flat, flat.reshape(-1, 2 * DIM), codebook)
    return (q.reshape(inputs.shape), loss[0, 0], perp[0, 0],
            idx3.reshape(-1))


# speedup vs baseline: 1.0881x; 1.0881x over previous
import jax
import jax.numpy as jnp
from jax import lax
from jax.experimental import pallas as pl
from jax.experimental.pallas import tpu as pltpu

NUM_EMB = 1024
DIM = 64
COMMIT = 0.25
TILE_N = 1152


def _vq_body(x_ref, cb_ref, q_ref, idx_ref, loss_ref, perp_ref,
             cb2h_ref, fiota_ref, counts_ref, lsum_ref):
    step = pl.program_id(0)
    nsteps = pl.num_programs(0)
    x = x_ref[...]
    cb = cb_ref[...]

    @pl.when(step == 0)
    def _():
        cb2h_ref[...] = 0.5 * jnp.sum(cb * cb, axis=1, keepdims=True)
        fiota_ref[...] = lax.broadcasted_iota(
            jnp.int32, (NUM_EMB, 1), 0).astype(jnp.float32)

    xcT = lax.dot_general(cb, x, (((1,), (1,)), ((), ())),
                          preferred_element_type=jnp.float32)
    sT = xcT - cb2h_ref[...]
    smax = jnp.max(sT, axis=0, keepdims=True)
    fiota = fiota_ref[...]
    idx_row = jnp.min(jnp.where(sT == smax, fiota, 131072.0),
                      axis=0, keepdims=True)
    idx_ref[0, :, :] = idx_row.astype(jnp.int32)
    onehotT = (fiota == idx_row).astype(jnp.float32)
    q_ref[...] = lax.dot_general(onehotT, cb, (((0,), (0,)), ((), ())),
                                 preferred_element_type=jnp.float32)
    part_loss = jnp.sum(x * x) - 2.0 * jnp.sum(smax)
    ones_col = jnp.ones((TILE_N, 1), jnp.float32)
    part_counts = lax.dot_general(onehotT, ones_col, (((1,), (0,)), ((), ())),
                                  preferred_element_type=jnp.float32)

    @pl.when(step == 0)
    def _():
        counts_ref[...] = part_counts
        lsum_ref[0] = part_loss

    @pl.when(step != 0)
    def _():
        counts_ref[...] += part_counts
        lsum_ref[0] += part_loss

    @pl.when(step == nsteps - 1)
    def _():
        n_total = nsteps * TILE_N
        p = counts_ref[...] * (1.0 / n_total)
        perp_ref[0, 0] = jnp.exp(-jnp.sum(p * jnp.log(p + 1e-10)))
        loss_ref[0, 0] = (1.0 + COMMIT) * lsum_ref[0] / (n_total * DIM)


def kernel(inputs, codebook):
    flat = inputs.reshape(-1, DIM)
    n = flat.shape[0]
    grid = (n // TILE_N,)
    q, idx3, loss, perp = pl.pallas_call(
        _vq_body,
        grid=grid,
        in_specs=[
            pl.BlockSpec((TILE_N, DIM), lambda i: (i, 0)),
            pl.BlockSpec((NUM_EMB, DIM), lambda i: (0, 0)),
        ],
        out_specs=[
            pl.BlockSpec((TILE_N, DIM), lambda i: (i, 0)),
            pl.BlockSpec((1, 1, TILE_N), lambda i: (i, 0, 0)),
            pl.BlockSpec(memory_space=pltpu.SMEM),
            pl.BlockSpec(memory_space=pltpu.SMEM),
        ],
        out_shape=[
            jax.ShapeDtypeStruct((n, DIM), jnp.float32),
            jax.ShapeDtypeStruct((n // TILE_N, 1, TILE_N), jnp.int32),
            jax.ShapeDtypeStruct((1, 1), jnp.float32),
            jax.ShapeDtypeStruct((1, 1), jnp.float32),
        ],
        scratch_shapes=[
            pltpu.VMEM((NUM_EMB, 1), jnp.float32),
            pltpu.VMEM((NUM_EMB, 1), jnp.float32),
            pltpu.VMEM((NUM_EMB, 1), jnp.float32),
            pltpu.SMEM((1,), jnp.float32),
        ],
        compiler_params=pltpu.CompilerParams(
            dimension_semantics=("arbitrary",)),
    )(flat, codebook)
    return (q.reshape(inputs.shape), loss[0, 0], perp[0, 0],
            idx3.reshape(-1))


# speedup vs baseline: 1.0957x; 1.0070x over previous
import jax
import jax.numpy as jnp
from jax import lax
from jax.experimental import pallas as pl
from jax.experimental.pallas import tpu as pltpu

NUM_EMB = 1024
DIM = 64
COMMIT = 0.25
TILE_N = 4608


def _vq_body(x_ref, cb_ref, q_ref, idx_ref, loss_ref, perp_ref,
             cb2h_ref, fiota_ref, counts_ref, lsum_ref):
    step = pl.program_id(0)
    nsteps = pl.num_programs(0)
    x = x_ref[...]
    cb = cb_ref[...]

    @pl.when(step == 0)
    def _():
        cb2h_ref[...] = 0.5 * jnp.sum(cb * cb, axis=1, keepdims=True)
        fiota_ref[...] = lax.broadcasted_iota(
            jnp.int32, (NUM_EMB, 1), 0).astype(jnp.float32)

    xcT = lax.dot_general(cb, x, (((1,), (1,)), ((), ())),
                          preferred_element_type=jnp.float32)
    sT = xcT - cb2h_ref[...]
    smax = jnp.max(sT, axis=0, keepdims=True)
    fiota = fiota_ref[...]
    idx_row = jnp.min(jnp.where(sT == smax, fiota, 131072.0),
                      axis=0, keepdims=True)
    idx_ref[0, :, :] = idx_row.astype(jnp.int32)
    onehotT = (fiota == idx_row).astype(jnp.float32)
    q_ref[...] = lax.dot_general(onehotT, cb, (((0,), (0,)), ((), ())),
                                 preferred_element_type=jnp.float32)
    part_loss = jnp.sum(x * x) - 2.0 * jnp.sum(smax)
    ones_col = jnp.ones((TILE_N, 1), jnp.float32)
    part_counts = lax.dot_general(onehotT, ones_col, (((1,), (0,)), ((), ())),
                                  preferred_element_type=jnp.float32)

    @pl.when(step == 0)
    def _():
        counts_ref[...] = part_counts
        lsum_ref[0] = part_loss

    @pl.when(step != 0)
    def _():
        counts_ref[...] += part_counts
        lsum_ref[0] += part_loss

    @pl.when(step == nsteps - 1)
    def _():
        n_total = nsteps * TILE_N
        p = counts_ref[...] * (1.0 / n_total)
        perp_ref[0, 0] = jnp.exp(-jnp.sum(p * jnp.log(p + 1e-10)))
        loss_ref[0, 0] = (1.0 + COMMIT) * lsum_ref[0] / (n_total * DIM)


def kernel(inputs, codebook):
    flat = inputs.reshape(-1, DIM)
    n = flat.shape[0]
    grid = (n // TILE_N,)
    q, idx3, loss, perp = pl.pallas_call(
        _vq_body,
        grid=grid,
        in_specs=[
            pl.BlockSpec((TILE_N, DIM), lambda i: (i, 0)),
            pl.BlockSpec((NUM_EMB, DIM), lambda i: (0, 0)),
        ],
        out_specs=[
            pl.BlockSpec((TILE_N, DIM), lambda i: (i, 0)),
            pl.BlockSpec((1, 1, TILE_N), lambda i: (i, 0, 0)),
            pl.BlockSpec(memory_space=pltpu.SMEM),
            pl.BlockSpec(memory_space=pltpu.SMEM),
        ],
        out_shape=[
            jax.ShapeDtypeStruct((n, DIM), jnp.float32),
            jax.ShapeDtypeStruct((n // TILE_N, 1, TILE_N), jnp.int32),
            jax.ShapeDtypeStruct((1, 1), jnp.float32),
            jax.ShapeDtypeStruct((1, 1), jnp.float32),
        ],
        scratch_shapes=[
            pltpu.VMEM((NUM_EMB, 1), jnp.float32),
            pltpu.VMEM((NUM_EMB, 1), jnp.float32),
            pltpu.VMEM((NUM_EMB, 1), jnp.float32),
            pltpu.SMEM((1,), jnp.float32),
        ],
        compiler_params=pltpu.CompilerParams(
            dimension_semantics=("arbitrary",)),
    )(flat, codebook)
    return (q.reshape(inputs.shape), loss[0, 0], perp[0, 0],
            idx3.reshape(-1))


# speedup vs baseline: 1.1180x; 1.0204x over previous
import jax
import jax.numpy as jnp
from jax import lax
from jax.experimental import pallas as pl
from jax.experimental.pallas import tpu as pltpu

NUM_EMB = 1024
DIM = 64
COMMIT = 0.25
TILE_N = 2304


def _vq_body(x_ref, cb_ref, q_ref, idx_ref, loss_ref, perp_ref,
             cb2h_ref, fiota_ref, cbb_ref, counts_ref, lsum_ref):
    step = pl.program_id(0)
    nsteps = pl.num_programs(0)
    x = x_ref[...]
    cb = cb_ref[...]

    @pl.when(step == 0)
    def _():
        cb2h_ref[...] = 0.5 * jnp.sum(cb * cb, axis=1, keepdims=True)
        fiota_ref[...] = lax.broadcasted_iota(
            jnp.int32, (NUM_EMB, 1), 0).astype(jnp.float32)
        cbb_ref[...] = cb.astype(jnp.bfloat16)

    xcT = lax.dot_general(cb, x, (((1,), (1,)), ((), ())),
                          preferred_element_type=jnp.float32)
    sT = xcT - cb2h_ref[...]
    smax = jnp.max(sT, axis=0, keepdims=True)
    fiota = fiota_ref[...]
    idx_row = jnp.min(jnp.where(sT == smax, fiota, 131072.0),
                      axis=0, keepdims=True)
    idx_ref[0, :, :] = idx_row.astype(jnp.int32)
    onehotT = (fiota == idx_row).astype(jnp.bfloat16)
    q_ref[...] = lax.dot_general(onehotT, cbb_ref[...],
                                 (((0,), (0,)), ((), ())),
                                 preferred_element_type=jnp.float32)
    part_loss = jnp.sum(x * x) - 2.0 * jnp.sum(smax)
    ones_col = jnp.ones((TILE_N, 1), jnp.bfloat16)
    part_counts = lax.dot_general(onehotT, ones_col, (((1,), (0,)), ((), ())),
                                  preferred_element_type=jnp.float32)

    @pl.when(step == 0)
    def _():
        counts_ref[...] = part_counts
        lsum_ref[0] = part_loss

    @pl.when(step != 0)
    def _():
        counts_ref[...] += part_counts
        lsum_ref[0] += part_loss

    @pl.when(step == nsteps - 1)
    def _():
        n_total = nsteps * TILE_N
        p = counts_ref[...] * (1.0 / n_total)
        perp_ref[0, 0] = jnp.exp(-jnp.sum(p * jnp.log(p + 1e-10)))
        loss_ref[0, 0] = (1.0 + COMMIT) * lsum_ref[0] / (n_total * DIM)


def kernel(inputs, codebook):
    flat = inputs.reshape(-1, DIM)
    n = flat.shape[0]
    grid = (n // TILE_N,)
    q, idx3, loss, perp = pl.pallas_call(
        _vq_body,
        grid=grid,
        in_specs=[
            pl.BlockSpec((TILE_N, DIM), lambda i: (i, 0)),
            pl.BlockSpec((NUM_EMB, DIM), lambda i: (0, 0)),
        ],
        out_specs=[
            pl.BlockSpec((TILE_N, DIM), lambda i: (i, 0)),
            pl.BlockSpec((1, 1, TILE_N), lambda i: (i, 0, 0)),
            pl.BlockSpec(memory_space=pltpu.SMEM),
            pl.BlockSpec(memory_space=pltpu.SMEM),
        ],
        out_shape=[
            jax.ShapeDtypeStruct((n, DIM), jnp.float32),
            jax.ShapeDtypeStruct((n // TILE_N, 1, TILE_N), jnp.int32),
            jax.ShapeDtypeStruct((1, 1), jnp.float32),
            jax.ShapeDtypeStruct((1, 1), jnp.float32),
        ],
        scratch_shapes=[
            pltpu.VMEM((NUM_EMB, 1), jnp.float32),
            pltpu.VMEM((NUM_EMB, 1), jnp.float32),
            pltpu.VMEM((NUM_EMB, DIM), jnp.bfloat16),
            pltpu.VMEM((NUM_EMB, 1), jnp.float32),
            pltpu.SMEM((1,), jnp.float32),
        ],
        compiler_params=pltpu.CompilerParams(
            dimension_semantics=("arbitrary",)),
    )(flat, codebook)
    return (q.reshape(inputs.shape), loss[0, 0], perp[0, 0],
            idx3.reshape(-1))


# speedup vs baseline: 1.1192x; 1.0011x over previous
import jax
import jax.numpy as jnp
from jax import lax
from jax.experimental import pallas as pl
from jax.experimental.pallas import tpu as pltpu

NUM_EMB = 1024
DIM = 64
COMMIT = 0.25
TILE_N = 2304


def _vq_body(x_ref, cb_ref, q_ref, idx_ref, loss_ref, perp_ref,
             cb2h_ref, fiota_ref, counts_ref, lsum_ref):
    step = pl.program_id(0)
    nsteps = pl.num_programs(0)
    x = x_ref[...]
    cb = cb_ref[...]

    @pl.when(step == 0)
    def _():
        cb2h_ref[...] = 0.5 * jnp.sum(cb * cb, axis=1, keepdims=True)
        fiota_ref[...] = lax.broadcasted_iota(
            jnp.int32, (NUM_EMB, 1), 0).astype(jnp.float32)

    xcT = lax.dot_general(cb, x, (((1,), (1,)), ((), ())),
                          preferred_element_type=jnp.float32)
    sT = xcT - cb2h_ref[...]
    smax = jnp.max(sT, axis=0, keepdims=True)
    fiota = fiota_ref[...]
    idx_row = jnp.min(jnp.where(sT == smax, fiota, 131072.0),
                      axis=0, keepdims=True)
    idx_ref[0, :, :] = idx_row.astype(jnp.int32)
    onehotT = (fiota == idx_row).astype(jnp.float32)
    q_ref[...] = lax.dot_general(onehotT, cb, (((0,), (0,)), ((), ())),
                                 preferred_element_type=jnp.float32)
    part_loss = jnp.sum(x * x) - 2.0 * jnp.sum(smax)
    ones_col = jnp.ones((TILE_N, 1), jnp.float32)
    part_counts = lax.dot_general(onehotT, ones_col, (((1,), (0,)), ((), ())),
                                  preferred_element_type=jnp.float32)

    @pl.when(step == 0)
    def _():
        counts_ref[...] = part_counts
        lsum_ref[0] = part_loss

    @pl.when(step != 0)
    def _():
        counts_ref[...] += part_counts
        lsum_ref[0] += part_loss

    @pl.when(step == nsteps - 1)
    def _():
        n_total = nsteps * TILE_N
        p = counts_ref[...] * (1.0 / n_total)
        perp_ref[0, 0] = jnp.exp(-jnp.sum(p * jnp.log(p + 1e-10)))
        loss_ref[0, 0] = (1.0 + COMMIT) * lsum_ref[0] / (n_total * DIM)


def kernel(inputs, codebook):
    flat = inputs.reshape(-1, DIM)
    n = flat.shape[0]
    grid = (n // TILE_N,)
    q, idx3, loss, perp = pl.pallas_call(
        _vq_body,
        grid=grid,
        in_specs=[
            pl.BlockSpec((TILE_N, DIM), lambda i: (i, 0)),
            pl.BlockSpec((NUM_EMB, DIM), lambda i: (0, 0)),
        ],
        out_specs=[
            pl.BlockSpec((TILE_N, DIM), lambda i: (i, 0)),
            pl.BlockSpec((1, 1, TILE_N), lambda i: (i, 0, 0)),
            pl.BlockSpec(memory_space=pltpu.SMEM),
            pl.BlockSpec(memory_space=pltpu.SMEM),
        ],
        out_shape=[
            jax.ShapeDtypeStruct((n, DIM), jnp.float32),
            jax.ShapeDtypeStruct((n // TILE_N, 1, TILE_N), jnp.int32),
            jax.ShapeDtypeStruct((1, 1), jnp.float32),
            jax.ShapeDtypeStruct((1, 1), jnp.float32),
        ],
        scratch_shapes=[
            pltpu.VMEM((NUM_EMB, 1), jnp.float32),
            pltpu.VMEM((NUM_EMB, 1), jnp.float32),
            pltpu.VMEM((NUM_EMB, 1), jnp.float32),
            pltpu.SMEM((1,), jnp.float32),
        ],
        compiler_params=pltpu.CompilerParams(
            dimension_semantics=("arbitrary",)),
    )(flat, codebook)
    return (q.reshape(inputs.shape), loss[0, 0], perp[0, 0],
            idx3.reshape(-1))
